# Initial kernel scaffold; baseline (speedup 1.0000x reference)
import jax
import jax.numpy as jnp
from jax.experimental import pallas as pl


def kernel(coords, features, W, b, gamma, beta, neighbor_indices):
    raise NotImplementedError("write your pallas kernel here")


# speedup vs baseline: 11.7353x; 11.7353x over previous
import functools

import jax
import jax.numpy as jnp
from jax import lax
from jax.experimental import pallas as pl
from jax.experimental.pallas import tpu as pltpu
from jax.experimental.pallas import tpu_sc as plsc

_B, _N, _K, _D = 2, 50000, 16, 16
_NK = _N * _K
_NW = 32
_ROWS_W = _NK // _NW
_CH = 1000
_NCHUNK = _ROWS_W // _CH

_RB = 6400
_NBLK = _RB // _K
_GRID_J = _NK // _RB


def _sc_gather(ct, ft, idx):
    mesh = plsc.VectorSubcoreMesh(core_axis_name="c", subcore_axis_name="s")

    @functools.partial(
        pl.kernel,
        mesh=mesh,
        out_type=[
            jax.ShapeDtypeStruct((_B, _NK, 16), jnp.float32),
            jax.ShapeDtypeStruct((_B, _NK, 16), jnp.float32),
        ],
        scratch_types=[
            pltpu.VMEM((_CH,), jnp.int32),
            pltpu.VMEM((_CH, 16), jnp.float32),
            pltpu.VMEM((_CH, 16), jnp.float32),
            pltpu.SemaphoreType.DMA,
        ],
        compiler_params=pltpu.CompilerParams(use_tc_tiling_on_sc=False),
    )
    def k(ct_hbm, ft_hbm, idx_hbm, gc_hbm, gf_hbm, idx_v, c_v, f_v, sem):
        wid = lax.axis_index("s") * 2 + lax.axis_index("c")
        for b in range(_B):
            for t in range(_NCHUNK):
                base = wid * _ROWS_W + t * _CH
                pltpu.sync_copy(idx_hbm.at[pl.ds(b * _NK + base, _CH)], idx_v)
                pltpu.async_copy(ct_hbm.at[b].at[idx_v], c_v, sem).wait()
                pltpu.sync_copy(c_v, gc_hbm.at[b, pl.ds(base, _CH)])
                pltpu.async_copy(ft_hbm.at[b].at[idx_v], f_v, sem).wait()
                pltpu.sync_copy(f_v, gf_hbm.at[b, pl.ds(base, _CH)])

    return k(ct, ft, idx)


def _x_block(cc, gc, wd, wcat, brow):
    ext = jnp.broadcast_to(
        cc.reshape(_NBLK, 1, 16), (_NBLK, _K, 16)
    ).reshape(_RB, 16)
    rp = ext - gc
    dist = jnp.sqrt(jnp.sum(rp * rp, axis=1, keepdims=True))
    rf = jnp.concatenate([rp, ext, gc], axis=1)
    x = lax.dot_general(
        rf, wcat, (((1,), (0,)), ((), ())), preferred_element_type=jnp.float32
    )
    return x + dist * wd + brow


def _stats_body(cc_ref, gc_ref, wd_ref, wc_ref, b_ref, out_ref, acc_ref):
    bi = pl.program_id(0)
    j = pl.program_id(1)

    @pl.when(jnp.logical_and(bi == 0, j == 0))
    def _():
        acc_ref[...] = jnp.zeros_like(acc_ref)

    x = _x_block(cc_ref[0], gc_ref[0], wd_ref[...], wc_ref[...], b_ref[...])
    acc_ref[0:1, :] += jnp.sum(x, axis=0, keepdims=True)
    acc_ref[1:2, :] += jnp.sum(x * x, axis=0, keepdims=True)

    @pl.when(jnp.logical_and(bi == _B - 1, j == _GRID_J - 1))
    def _():
        out_ref[...] = acc_ref[...]


def _out_body(cc_ref, gc_ref, gf_ref, wd_ref, wc_ref, b_ref, out_ref):
    x = _x_block(cc_ref[0], gc_ref[0], wd_ref[...], wc_ref[...], b_ref[...])
    x = jnp.maximum(x, 0.0)
    rows = jnp.concatenate([gf_ref[0], x], axis=1)
    eye = jnp.eye(32, dtype=jnp.float32)
    out_ref[0] = lax.dot_general(
        eye, rows, (((1,), (1,)), ((), ())), preferred_element_type=jnp.float32
    )


_SMALL_SPECS = [
    pl.BlockSpec((1, 16), lambda b, j: (0, 0)),
    pl.BlockSpec((48, 16), lambda b, j: (0, 0)),
    pl.BlockSpec((1, 16), lambda b, j: (0, 0)),
]


def _stats_pass(cc, gc, wd, wcat, brow):
    return pl.pallas_call(
        _stats_body,
        grid=(_B, _GRID_J),
        in_specs=[
            pl.BlockSpec((1, _NBLK, 16), lambda b, j: (b, j, 0)),
            pl.BlockSpec((1, _RB, 16), lambda b, j: (b, j, 0)),
        ] + _SMALL_SPECS,
        out_specs=pl.BlockSpec((2, 16), lambda b, j: (0, 0)),
        out_shape=jax.ShapeDtypeStruct((2, 16), jnp.float32),
        scratch_shapes=[pltpu.VMEM((2, 16), jnp.float32)],
    )(cc, gc, wd, wcat, brow)


def _out_pass(cc, gc, gf, wd, wcat, brow):
    return pl.pallas_call(
        _out_body,
        grid=(_B, _GRID_J),
        in_specs=[
            pl.BlockSpec((1, _NBLK, 16), lambda b, j: (b, j, 0)),
            pl.BlockSpec((1, _RB, 16), lambda b, j: (b, j, 0)),
            pl.BlockSpec((1, _RB, 16), lambda b, j: (b, j, 0)),
        ] + _SMALL_SPECS,
        out_specs=pl.BlockSpec((1, 32, _RB), lambda b, j: (b, 0, j)),
        out_shape=jax.ShapeDtypeStruct((_B, 32, _NK), jnp.float32),
    )(cc, gc, gf, wd, wcat, brow)


def kernel(coords, features, W, b, gamma, beta, neighbor_indices):
    ct = jnp.pad(coords, ((0, 0), (0, 0), (0, 13)))
    ft = jnp.transpose(features[..., 0], (0, 2, 1))
    idx = neighbor_indices.reshape(_B * _NK)

    gc, gf = _sc_gather(ct, ft, idx)

    Wt = W.T.astype(jnp.float32)
    wd = Wt[0:1]
    z = jnp.zeros((13, 16), jnp.float32)
    wcat = jnp.concatenate(
        [Wt[1:4], z, Wt[4:7], z, Wt[7:10], z], axis=0
    )
    brow = b.reshape(1, 16).astype(jnp.float32)

    sums = _stats_pass(ct, gc, wd, wcat, brow)
    m = float(_B * _NK)
    mean = sums[0] / m
    var = sums[1] / m - mean * mean
    scale = gamma / jnp.sqrt(var + 1e-6)
    shift = beta - mean * scale
    wd2 = wd * scale[None, :]
    wcat2 = wcat * scale[None, :]
    b2 = brow * scale[None, :] + shift[None, :]

    out = _out_pass(ct, gc, gf, wd2, wcat2, b2)
    return out.reshape(_B, 2 * _D, _N, _K)
